# Initial kernel scaffold; baseline (speedup 1.0000x reference)
import jax
import jax.numpy as jnp
from jax.experimental import pallas as pl


def kernel(seed, gt_s):
    raise NotImplementedError("write your pallas kernel here")


# speedup vs baseline: 2.8355x; 2.8355x over previous
import functools

import jax
import jax.numpy as jnp
from jax import lax
from jax.experimental import pallas as pl
from jax.experimental.pallas import tpu as pltpu
from jax.experimental.pallas import tpu_sc as plsc

NC, NS, L = 2, 16, 16
NW = NC * NS
N = 2048
B = 8
HALF = N // 2
NBLK = N // L
K = 16


def _round_bf16(v):
    u = plsc.bitcast(v, jnp.uint32)
    r = (u + jnp.uint32(0x7FFF) + ((u >> jnp.uint32(16)) & jnp.uint32(1)))
    r = r & jnp.uint32(0xFFFF0000)
    return plsc.bitcast(r, jnp.float32)


def _knn_body(pts_hbm, out_hbm, cand_v, candr_v, cc_v, acc_v):
    wid = lax.axis_index("s") * NC + lax.axis_index("c")
    prob = wid // 2
    half = wid % 2
    pltpu.sync_copy(pts_hbm.at[prob], cand_v)

    inf_v = jnp.full((L,), jnp.inf, dtype=jnp.float32)
    zero_v = jnp.zeros((L,), dtype=jnp.float32)

    def pre_body(j, carry):
        base = j * L
        cx = cand_v[0, pl.ds(base, L)]
        cy = cand_v[1, pl.ds(base, L)]
        cz = cand_v[2, pl.ds(base, L)]
        candr_v[0, pl.ds(base, L)] = _round_bf16(cx)
        candr_v[1, pl.ds(base, L)] = _round_bf16(cy)
        candr_v[2, pl.ds(base, L)] = _round_bf16(cz)
        cc_v[pl.ds(base, L)] = (cx * cx + cy * cy) + cz * cz
        return carry

    lax.fori_loop(0, NBLK, pre_body, 0)

    def qblk_body(qb, acc_outer):
        qbase = half * HALF + qb * L
        qxb = cand_v[0, pl.ds(qbase, L)]
        qyb = cand_v[1, pl.ds(qbase, L)]
        qzb = cand_v[2, pl.ds(qbase, L)]
        qqb = (qxb * qxb + qyb * qyb) + qzb * qzb
        qxrb = _round_bf16(qxb)
        qyrb = _round_bf16(qyb)
        qzrb = _round_bf16(qzb)

        def make_row(lane):
            qq = jnp.full((L,), qqb[lane], dtype=jnp.float32)
            qx = jnp.full((L,), qxrb[lane], dtype=jnp.float32)
            qy = jnp.full((L,), qyrb[lane], dtype=jnp.float32)
            qz = jnp.full((L,), qzrb[lane], dtype=jnp.float32)

            def blk_body(j, carry):
                top, thr = carry
                base = j * L
                inner = (candr_v[0, pl.ds(base, L)] * qx
                         + candr_v[1, pl.ds(base, L)] * qy)
                inner = inner + candr_v[2, pl.ds(base, L)] * qz
                d = (qq - (inner + inner)) + cc_v[pl.ds(base, L)]
                d = jnp.maximum(d, zero_v)

                def merge(args):
                    top, _, d = args
                    c_desc, _ = plsc.sort_key_val(d, d, descending=True)
                    lo = jnp.minimum(top, c_desc)
                    top_n, _ = plsc.sort_key_val(lo, lo)
                    thr_n = jnp.full((L,), top_n[L - 1], dtype=jnp.float32)
                    return top_n, thr_n

                nbeat = plsc.all_reduce_population_count(d < thr)
                return lax.cond(nbeat[0] > 0, merge,
                                lambda args: (args[0], args[1]), (top, thr, d))

            top, _ = lax.fori_loop(0, NBLK, blk_body, (inf_v, inf_v))
            return top

        acc = acc_outer
        for lane in range(L):
            acc = acc + make_row(lane)
        return acc

    acc = lax.fori_loop(0, HALF // L, qblk_body,
                        jnp.zeros((L,), dtype=jnp.float32))
    acc_v[...] = acc
    pltpu.sync_copy(acc_v, out_hbm.at[wid])


_knn = functools.partial(
    pl.kernel,
    out_type=jax.ShapeDtypeStruct((NW, L), jnp.float32),
    mesh=plsc.VectorSubcoreMesh(core_axis_name="c", subcore_axis_name="s",
                                num_cores=NC, num_subcores=NS),
    scratch_types=[
        pltpu.VMEM((3, N), jnp.float32),
        pltpu.VMEM((3, N), jnp.float32),
        pltpu.VMEM((N,), jnp.float32),
        pltpu.VMEM((L,), jnp.float32),
    ],
    compiler_params=pltpu.CompilerParams(needs_layout_passes=False),
)(_knn_body)


def kernel(seed, gt_s):
    pts = jnp.stack([seed, gt_s])
    pts = pts.transpose(0, 1, 3, 2).reshape(2 * B, 3, N)
    out = _knn(pts)
    per_prob = out.sum(axis=1).reshape(2 * B, 2).sum(axis=1)
    means = (per_prob / (N * K)).reshape(2, B)
    return jnp.mean((means[0] - means[1]) ** 2)
